# baseline (speedup 1.0000x reference)
import functools

import jax
import jax.numpy as jnp
from jax.experimental import pallas as pl
from jax.experimental.pallas import tpu as pltpu

_B, _S = 128, 64
_D = 64
_P = 512
_K = 8192
_N = _B * _S
_EPS = 0.1

_R = 1024
_KT = 512


def _fused_body(x_ref, c_ref, o_ref, w_ref, b_ref, mask_ref, out_ref,
                m_ref, s_ref, ps_ref, dmin_ref, bp_ref, accn_ref, accd_ref):
    i = pl.program_id(0)
    j = pl.program_id(1)
    ni = pl.num_programs(0)
    nj = pl.num_programs(1)

    @pl.when(jnp.logical_and(i == 0, j == 0))
    def _init_acc():
        accn_ref[...] = jnp.zeros_like(accn_ref)
        accd_ref[...] = jnp.zeros_like(accd_ref)

    @pl.when(j == 0)
    def _init_row_state():
        m_ref[...] = jnp.full_like(m_ref, -jnp.inf)
        s_ref[...] = jnp.zeros_like(s_ref)
        ps_ref[...] = jnp.zeros_like(ps_ref)
        dmin_ref[...] = jnp.full_like(dmin_ref, jnp.inf)
        bp_ref[...] = jnp.zeros_like(bp_ref)

    x = x_ref[...]
    c = c_ref[...]
    xc = jnp.dot(x, c, preferred_element_type=jnp.float32)
    c2 = jnp.sum(c * c, axis=0, keepdims=True)
    dist = c2 - 2.0 * xc

    o = o_ref[...]
    w = w_ref[...]
    pred = jax.lax.dot_general(o, w, (((1,), (1,)), ((), ())),
                               preferred_element_type=jnp.float32)
    pred = pred + b_ref[...]

    tmax = jnp.max(pred, axis=1, keepdims=True)
    m_old = m_ref[...]
    m_new = jnp.maximum(m_old, tmax)
    s_ref[...] = (s_ref[...] * jnp.exp(m_old - m_new)
                  + jnp.sum(jnp.exp(pred - m_new), axis=1, keepdims=True))
    m_ref[...] = m_new
    ps_ref[...] = ps_ref[...] + jnp.sum(pred, axis=1, keepdims=True)

    tmin = jnp.min(dist, axis=1, keepdims=True)
    pred_at = jnp.min(jnp.where(dist == tmin, pred, jnp.inf),
                      axis=1, keepdims=True)
    upd = tmin < dmin_ref[...]
    bp_ref[...] = jnp.where(upd, pred_at, bp_ref[...])
    dmin_ref[...] = jnp.minimum(dmin_ref[...], tmin)

    @pl.when(j == nj - 1)
    def _finish_rows():
        lse = m_ref[...] + jnp.log(s_ref[...])
        nll = lse - bp_ref[...]
        smooth = lse - ps_ref[...] * (1.0 / _K)
        elem = (1.0 - _EPS) * nll + _EPS * smooth
        mk = mask_ref[...]
        accn_ref[...] = accn_ref[...] + jnp.sum(mk * elem, keepdims=True)
        accd_ref[...] = accd_ref[...] + jnp.sum(mk, keepdims=True)

        @pl.when(i == ni - 1)
        def _emit():
            out_ref[...] = accn_ref[...] / accd_ref[...]


@functools.partial(jax.jit, static_argnames=("interpret",))
def _fused_loss(x, mask_f, o, centroids, head_w, head_b, interpret=False):
    grid = (_N // _R, _K // _KT)
    out = pl.pallas_call(
        _fused_body,
        grid=grid,
        in_specs=[
            pl.BlockSpec((_R, _D), lambda i, j: (i, 0)),
            pl.BlockSpec((_D, _KT), lambda i, j: (0, j)),
            pl.BlockSpec((_R, _P), lambda i, j: (i, 0)),
            pl.BlockSpec((_KT, _P), lambda i, j: (j, 0)),
            pl.BlockSpec((1, _KT), lambda i, j: (0, j)),
            pl.BlockSpec((_R, 1), lambda i, j: (i, 0)),
        ],
        out_specs=pl.BlockSpec((1, 1), lambda i, j: (0, 0)),
        out_shape=jax.ShapeDtypeStruct((1, 1), jnp.float32),
        scratch_shapes=[
            pltpu.VMEM((_R, 1), jnp.float32),
            pltpu.VMEM((_R, 1), jnp.float32),
            pltpu.VMEM((_R, 1), jnp.float32),
            pltpu.VMEM((_R, 1), jnp.float32),
            pltpu.VMEM((_R, 1), jnp.float32),
            pltpu.VMEM((1, 1), jnp.float32),
            pltpu.VMEM((1, 1), jnp.float32),
        ],
        compiler_params=pltpu.CompilerParams(
            dimension_semantics=("arbitrary", "arbitrary"),
        ),
        interpret=interpret,
    )(x, centroids, o, head_w, head_b, mask_f)
    return out[0, 0]


def kernel(csts, null_mask, outputs, centroids, head_w, head_b):
    x = csts.reshape(_N, _D)
    o = outputs.reshape(_N, _P).astype(jnp.bfloat16)
    head_w = head_w.astype(jnp.bfloat16)
    mask_f = null_mask.reshape(_N, 1).astype(jnp.float32)
    b2 = head_b.reshape(1, _K)
    return _fused_loss(x, mask_f, o, centroids, head_w, b2)


# speedup vs baseline: 1.2670x; 1.2670x over previous
import functools

import jax
import jax.numpy as jnp
from jax.experimental import pallas as pl
from jax.experimental.pallas import tpu as pltpu

_B, _S = 128, 64
_D = 64
_P = 512
_K = 8192
_N = _B * _S
_EPS = 0.1

_R = 1024
_KT = 1024


def _fused_body(x_ref, c_ref, o_ref, w_ref, b_ref, mask_ref, out_ref,
                m_ref, s_ref, ps_ref, dmin_ref, bp_ref, accn_ref, accd_ref):
    i = pl.program_id(0)
    j = pl.program_id(1)
    ni = pl.num_programs(0)
    nj = pl.num_programs(1)

    @pl.when(jnp.logical_and(i == 0, j == 0))
    def _init_acc():
        accn_ref[...] = jnp.zeros_like(accn_ref)
        accd_ref[...] = jnp.zeros_like(accd_ref)

    @pl.when(j == 0)
    def _init_row_state():
        m_ref[...] = jnp.full_like(m_ref, -jnp.inf)
        s_ref[...] = jnp.zeros_like(s_ref)
        ps_ref[...] = jnp.zeros_like(ps_ref)
        dmin_ref[...] = jnp.full_like(dmin_ref, jnp.inf)
        bp_ref[...] = jnp.zeros_like(bp_ref)

    x = x_ref[...]
    c = c_ref[...]
    xc = jnp.dot(x, c, preferred_element_type=jnp.float32)
    c2 = jnp.sum(c * c, axis=0, keepdims=True)
    dist = c2 - 2.0 * xc

    o = o_ref[...]
    w = w_ref[...]
    pred = jax.lax.dot_general(o, w, (((1,), (1,)), ((), ())),
                               preferred_element_type=jnp.float32)
    pred = pred + b_ref[...]

    ones = jnp.ones((_KT, 1), jnp.float32)
    _rowsum = lambda t: jax.lax.dot_general(
        t, ones, (((1,), (0,)), ((), ())), preferred_element_type=jnp.float32)

    tmax = jnp.max(pred, axis=1, keepdims=True)
    m_old = m_ref[...]
    m_new = jnp.maximum(m_old, tmax)
    e = jnp.exp(pred - m_new)
    s_ref[...] = s_ref[...] * jnp.exp(m_old - m_new) + _rowsum(e)
    m_ref[...] = m_new
    ps_ref[...] = ps_ref[...] + _rowsum(pred)

    tmin = jnp.min(dist, axis=1, keepdims=True)
    pred_at = _rowsum(jnp.where(dist == tmin, pred, 0.0))
    upd = tmin < dmin_ref[...]
    bp_ref[...] = jnp.where(upd, pred_at, bp_ref[...])
    dmin_ref[...] = jnp.minimum(dmin_ref[...], tmin)

    @pl.when(j == nj - 1)
    def _finish_rows():
        lse = m_ref[...] + jnp.log(s_ref[...])
        nll = lse - bp_ref[...]
        smooth = lse - ps_ref[...] * (1.0 / _K)
        elem = (1.0 - _EPS) * nll + _EPS * smooth
        mk = mask_ref[...]
        accn_ref[...] = accn_ref[...] + jnp.sum(mk * elem, keepdims=True)
        accd_ref[...] = accd_ref[...] + jnp.sum(mk, keepdims=True)

        @pl.when(i == ni - 1)
        def _emit():
            out_ref[...] = accn_ref[...] / accd_ref[...]


@functools.partial(jax.jit, static_argnames=("interpret",))
def _fused_loss(x, mask_f, o, centroids, head_w, head_b, interpret=False):
    grid = (_N // _R, _K // _KT)
    out = pl.pallas_call(
        _fused_body,
        grid=grid,
        in_specs=[
            pl.BlockSpec((_R, _D), lambda i, j: (i, 0)),
            pl.BlockSpec((_D, _KT), lambda i, j: (0, j)),
            pl.BlockSpec((_R, _P), lambda i, j: (i, 0)),
            pl.BlockSpec((_KT, _P), lambda i, j: (j, 0)),
            pl.BlockSpec((1, _KT), lambda i, j: (0, j)),
            pl.BlockSpec((_R, 1), lambda i, j: (i, 0)),
        ],
        out_specs=pl.BlockSpec((1, 1), lambda i, j: (0, 0)),
        out_shape=jax.ShapeDtypeStruct((1, 1), jnp.float32),
        scratch_shapes=[
            pltpu.VMEM((_R, 1), jnp.float32),
            pltpu.VMEM((_R, 1), jnp.float32),
            pltpu.VMEM((_R, 1), jnp.float32),
            pltpu.VMEM((_R, 1), jnp.float32),
            pltpu.VMEM((_R, 1), jnp.float32),
            pltpu.VMEM((1, 1), jnp.float32),
            pltpu.VMEM((1, 1), jnp.float32),
        ],
        compiler_params=pltpu.CompilerParams(
            dimension_semantics=("arbitrary", "arbitrary"),
        ),
        interpret=interpret,
    )(x, centroids, o, head_w, head_b, mask_f)
    return out[0, 0]


def kernel(csts, null_mask, outputs, centroids, head_w, head_b):
    x = csts.reshape(_N, _D)
    o = outputs.reshape(_N, _P).astype(jnp.bfloat16)
    head_w = head_w.astype(jnp.bfloat16)
    mask_f = null_mask.reshape(_N, 1).astype(jnp.float32)
    b2 = head_b.reshape(1, _K)
    return _fused_loss(x, mask_f, o, centroids, head_w, b2)


# speedup vs baseline: 1.2959x; 1.0228x over previous
import functools

import jax
import jax.numpy as jnp
from jax.experimental import pallas as pl
from jax.experimental.pallas import tpu as pltpu

_B, _S = 128, 64
_D = 64
_P = 512
_K = 8192
_N = _B * _S
_EPS = 0.1

_R = 1024
_KT = 2048


def _fused_body(x_ref, c_ref, o_ref, w_ref, b_ref, mask_ref, out_ref,
                m_ref, s_ref, ps_ref, dmin_ref, bp_ref, accn_ref, accd_ref):
    i = pl.program_id(0)
    j = pl.program_id(1)
    ni = pl.num_programs(0)
    nj = pl.num_programs(1)

    @pl.when(jnp.logical_and(i == 0, j == 0))
    def _init_acc():
        accn_ref[...] = jnp.zeros_like(accn_ref)
        accd_ref[...] = jnp.zeros_like(accd_ref)

    @pl.when(j == 0)
    def _init_row_state():
        m_ref[...] = jnp.full_like(m_ref, -jnp.inf)
        s_ref[...] = jnp.zeros_like(s_ref)
        ps_ref[...] = jnp.zeros_like(ps_ref)
        dmin_ref[...] = jnp.full_like(dmin_ref, jnp.inf)
        bp_ref[...] = jnp.zeros_like(bp_ref)

    x = x_ref[...]
    c = c_ref[...]
    xc = jnp.dot(x, c, preferred_element_type=jnp.float32)
    c2 = jnp.sum(c * c, axis=0, keepdims=True)
    dist = c2 - 2.0 * xc

    o = o_ref[...]
    w = w_ref[...]
    pred = jax.lax.dot_general(o, w, (((1,), (1,)), ((), ())),
                               preferred_element_type=jnp.float32)
    pred = pred + b_ref[...]

    ones = jnp.ones((_KT, 1), jnp.float32)
    _rowsum = lambda t: jax.lax.dot_general(
        t, ones, (((1,), (0,)), ((), ())), preferred_element_type=jnp.float32)

    tmax = jnp.max(pred, axis=1, keepdims=True)
    m_old = m_ref[...]
    m_new = jnp.maximum(m_old, tmax)
    e = jnp.exp(pred - m_new)
    s_ref[...] = s_ref[...] * jnp.exp(m_old - m_new) + _rowsum(e)
    m_ref[...] = m_new
    ps_ref[...] = ps_ref[...] + _rowsum(pred)

    tmin = jnp.min(dist, axis=1, keepdims=True)
    pred_at = _rowsum(jnp.where(dist == tmin, pred, 0.0))
    upd = tmin < dmin_ref[...]
    bp_ref[...] = jnp.where(upd, pred_at, bp_ref[...])
    dmin_ref[...] = jnp.minimum(dmin_ref[...], tmin)

    @pl.when(j == nj - 1)
    def _finish_rows():
        lse = m_ref[...] + jnp.log(s_ref[...])
        nll = lse - bp_ref[...]
        smooth = lse - ps_ref[...] * (1.0 / _K)
        elem = (1.0 - _EPS) * nll + _EPS * smooth
        mk = mask_ref[...]
        accn_ref[...] = accn_ref[...] + jnp.sum(mk * elem, keepdims=True)
        accd_ref[...] = accd_ref[...] + jnp.sum(mk, keepdims=True)

        @pl.when(i == ni - 1)
        def _emit():
            out_ref[...] = accn_ref[...] / accd_ref[...]


@functools.partial(jax.jit, static_argnames=("interpret",))
def _fused_loss(x, mask_f, o, centroids, head_w, head_b, interpret=False):
    grid = (_N // _R, _K // _KT)
    out = pl.pallas_call(
        _fused_body,
        grid=grid,
        in_specs=[
            pl.BlockSpec((_R, _D), lambda i, j: (i, 0)),
            pl.BlockSpec((_D, _KT), lambda i, j: (0, j)),
            pl.BlockSpec((_R, _P), lambda i, j: (i, 0)),
            pl.BlockSpec((_KT, _P), lambda i, j: (j, 0)),
            pl.BlockSpec((1, _KT), lambda i, j: (0, j)),
            pl.BlockSpec((_R, 1), lambda i, j: (i, 0)),
        ],
        out_specs=pl.BlockSpec((1, 1), lambda i, j: (0, 0)),
        out_shape=jax.ShapeDtypeStruct((1, 1), jnp.float32),
        scratch_shapes=[
            pltpu.VMEM((_R, 1), jnp.float32),
            pltpu.VMEM((_R, 1), jnp.float32),
            pltpu.VMEM((_R, 1), jnp.float32),
            pltpu.VMEM((_R, 1), jnp.float32),
            pltpu.VMEM((_R, 1), jnp.float32),
            pltpu.VMEM((1, 1), jnp.float32),
            pltpu.VMEM((1, 1), jnp.float32),
        ],
        compiler_params=pltpu.CompilerParams(
            dimension_semantics=("arbitrary", "arbitrary"),
        ),
        interpret=interpret,
    )(x, centroids, o, head_w, head_b, mask_f)
    return out[0, 0]


def kernel(csts, null_mask, outputs, centroids, head_w, head_b):
    x = csts.reshape(_N, _D)
    o = outputs.reshape(_N, _P).astype(jnp.bfloat16)
    head_w = head_w.astype(jnp.bfloat16)
    mask_f = null_mask.reshape(_N, 1).astype(jnp.float32)
    b2 = head_b.reshape(1, _K)
    return _fused_loss(x, mask_f, o, centroids, head_w, b2)


# speedup vs baseline: 1.4379x; 1.1095x over previous
import functools

import jax
import jax.numpy as jnp
from jax.experimental import pallas as pl
from jax.experimental.pallas import tpu as pltpu

_B, _S = 128, 64
_D = 64
_P = 512
_K = 8192
_N = _B * _S
_EPS = 0.1

_R = 1024
_KT = 2048


def _fused_body(x_ref, c_ref, o_ref, w_ref, b_ref, mask_ref, out_ref,
                m_ref, s_ref, dmin_ref, bp_ref, accn_ref, accd_ref,
                wsum_ref, bsum_ref):
    i = pl.program_id(0)
    j = pl.program_id(1)
    ni = pl.num_programs(0)
    nj = pl.num_programs(1)

    @pl.when(jnp.logical_and(i == 0, j == 0))
    def _init_acc():
        accn_ref[...] = jnp.zeros_like(accn_ref)
        accd_ref[...] = jnp.zeros_like(accd_ref)

    @pl.when(j == 0)
    def _init_row_state():
        m_ref[...] = jnp.full_like(m_ref, -jnp.inf)
        s_ref[...] = jnp.zeros_like(s_ref)
        dmin_ref[...] = jnp.full_like(dmin_ref, jnp.inf)
        bp_ref[...] = jnp.zeros_like(bp_ref)

    x = x_ref[...]
    c = c_ref[...]
    xc = jnp.dot(x, c, preferred_element_type=jnp.float32)
    c2 = jnp.sum(c * c, axis=0, keepdims=True)
    dist = c2 - 2.0 * xc

    o = o_ref[...]
    w = w_ref[...]
    pred = jax.lax.dot_general(o, w, (((1,), (1,)), ((), ())),
                               preferred_element_type=jnp.float32)
    pred = pred + b_ref[...]

    ones = jnp.ones((_KT, 1), jnp.float32)
    _rowsum = lambda t: jax.lax.dot_general(
        t, ones, (((1,), (0,)), ((), ())), preferred_element_type=jnp.float32)

    tmax = jnp.max(pred, axis=1, keepdims=True)
    m_old = m_ref[...]
    m_new = jnp.maximum(m_old, tmax)
    e = jnp.exp(pred - m_new)
    s_ref[...] = s_ref[...] * jnp.exp(m_old - m_new) + _rowsum(e)
    m_ref[...] = m_new

    @pl.when(i == 0)
    def _acc_wsum():
        wprev = jnp.where(j == 0, jnp.zeros_like(wsum_ref), wsum_ref[...])
        bprev = jnp.where(j == 0, jnp.zeros_like(bsum_ref), bsum_ref[...])
        wsum_ref[...] = wprev + jnp.sum(w.astype(jnp.float32), axis=0,
                                        keepdims=True)
        bsum_ref[...] = bprev + jnp.sum(b_ref[...], keepdims=True)

    tmin = jnp.min(dist, axis=1, keepdims=True)
    pred_at = _rowsum(jnp.where(dist == tmin, pred, 0.0))
    upd = tmin < dmin_ref[...]
    bp_ref[...] = jnp.where(upd, pred_at, bp_ref[...])
    dmin_ref[...] = jnp.minimum(dmin_ref[...], tmin)

    @pl.when(j == nj - 1)
    def _finish_rows():
        lse = m_ref[...] + jnp.log(s_ref[...])
        nll = lse - bp_ref[...]
        ow = o.astype(jnp.float32) * wsum_ref[...]
        ps = jax.lax.dot_general(
            ow, jnp.ones((_P, 1), jnp.float32), (((1,), (0,)), ((), ())),
            preferred_element_type=jnp.float32) + bsum_ref[...]
        smooth = lse - ps * (1.0 / _K)
        elem = (1.0 - _EPS) * nll + _EPS * smooth
        mk = mask_ref[...]
        accn_ref[...] = accn_ref[...] + jnp.sum(mk * elem, keepdims=True)
        accd_ref[...] = accd_ref[...] + jnp.sum(mk, keepdims=True)

        @pl.when(i == ni - 1)
        def _emit():
            out_ref[...] = accn_ref[...] / accd_ref[...]


@functools.partial(jax.jit, static_argnames=("interpret",))
def _fused_loss(x, mask_f, o, centroids, head_w, head_b, interpret=False):
    grid = (_N // _R, _K // _KT)
    out = pl.pallas_call(
        _fused_body,
        grid=grid,
        in_specs=[
            pl.BlockSpec((_R, _D), lambda i, j: (i, 0)),
            pl.BlockSpec((_D, _KT), lambda i, j: (0, j)),
            pl.BlockSpec((_R, _P), lambda i, j: (i, 0)),
            pl.BlockSpec((_KT, _P), lambda i, j: (j, 0)),
            pl.BlockSpec((1, _KT), lambda i, j: (0, j)),
            pl.BlockSpec((_R, 1), lambda i, j: (i, 0)),
        ],
        out_specs=pl.BlockSpec((1, 1), lambda i, j: (0, 0)),
        out_shape=jax.ShapeDtypeStruct((1, 1), jnp.float32),
        scratch_shapes=[
            pltpu.VMEM((_R, 1), jnp.float32),
            pltpu.VMEM((_R, 1), jnp.float32),
            pltpu.VMEM((_R, 1), jnp.float32),
            pltpu.VMEM((_R, 1), jnp.float32),
            pltpu.VMEM((1, 1), jnp.float32),
            pltpu.VMEM((1, 1), jnp.float32),
            pltpu.VMEM((1, _P), jnp.float32),
            pltpu.VMEM((1, 1), jnp.float32),
        ],
        compiler_params=pltpu.CompilerParams(
            dimension_semantics=("arbitrary", "arbitrary"),
        ),
        interpret=interpret,
    )(x, centroids, o, head_w, head_b, mask_f)
    return out[0, 0]


def kernel(csts, null_mask, outputs, centroids, head_w, head_b):
    x = csts.reshape(_N, _D)
    o = outputs.reshape(_N, _P).astype(jnp.bfloat16)
    head_w = head_w.astype(jnp.bfloat16)
    mask_f = null_mask.reshape(_N, 1).astype(jnp.float32)
    b2 = head_b.reshape(1, _K)
    return _fused_loss(x, mask_f, o, centroids, head_w, b2)
